# baseline (speedup 1.0000x reference)
import functools

import jax
import jax.numpy as jnp
import numpy as np
from jax import lax
from jax.experimental import pallas as pl
from jax.experimental.pallas import tpu as pltpu
from jax.experimental.pallas import tpu_sc as plsc

B, S, D = 4, 8192, 768
H = D // 2
OUT = 4096
K = 200
SEL = 208
SR, SC_ = 64, 128
INT_MIN = np.int32(-2**31)


_BS = 512


def _score_body(x_ref, w1_ref, b1_ref, w2_ref, k_ref):
    xb = x_ref[...].reshape(B * _BS, D)
    h = jnp.dot(xb, w1_ref[...], preferred_element_type=jnp.float32)
    h = h + b1_ref[...]
    h = 0.5 * h * (1.0 + lax.erf(h * np.float32(np.sqrt(0.5))))
    s = jnp.dot(h, w2_ref[...], preferred_element_type=jnp.float32)
    bits = lax.bitcast_convert_type(s[:, 0], jnp.int32)
    keys = jnp.where(bits < 0, bits ^ jnp.int32(0x7FFFFFFF), bits)
    k_ref[...] = keys.reshape(B, _BS)


def _scores(x, w1t, b1r, w2t):
    return pl.pallas_call(
        _score_body,
        grid=(S // _BS,),
        in_specs=[
            pl.BlockSpec((B, _BS, D), lambda s: (0, s, 0)),
            pl.BlockSpec((D, H), lambda s: (0, 0)),
            pl.BlockSpec((1, H), lambda s: (0, 0)),
            pl.BlockSpec((H, 1), lambda s: (0, 0)),
        ],
        out_specs=pl.BlockSpec((B, _BS), lambda s: (0, s)),
        out_shape=jax.ShapeDtypeStruct((B, S), jnp.int32),
    )(x, w1t, b1r, w2t)


def _prefix_incl(m):
    m2 = m.reshape(B * SR, SC_)
    tri_c = (lax.broadcasted_iota(jnp.int32, (SC_, SC_), 0)
             <= lax.broadcasted_iota(jnp.int32, (SC_, SC_), 1))
    intra = jnp.dot(m2, tri_c.astype(jnp.float32),
                    precision=lax.Precision.HIGHEST,
                    preferred_element_type=jnp.float32)
    totals = intra[:, SC_ - 1].reshape(B, SR)
    tri_r = (lax.broadcasted_iota(jnp.int32, (SR, SR), 0)
             < lax.broadcasted_iota(jnp.int32, (SR, SR), 1))
    rowpre = jnp.dot(totals, tri_r.astype(jnp.float32),
                     precision=lax.Precision.HIGHEST,
                     preferred_element_type=jnp.float32)
    out = intra.reshape(B, SR, SC_) + rowpre[:, :, None]
    return out.reshape(B, S)


def _select_body(k_ref, gidx_ref, selk_ref):
    keys = k_ref[...]

    def bs_body(i, p):
        cand_u = p | (jnp.int32(1) << (jnp.int32(31) - i))
        cand_s = cand_u ^ INT_MIN
        cnt = jnp.sum((keys >= cand_s).astype(jnp.int32),
                      axis=1, keepdims=True)
        return jnp.where(cnt >= K, cand_u, p)
    t_u = lax.fori_loop(0, 32, bs_body, jnp.zeros((B, 1), jnp.int32))
    t_s = t_u ^ INT_MIN

    mgt = keys > t_s
    meq = keys == t_s
    ngt = jnp.sum(mgt.astype(jnp.int32), axis=1, keepdims=True)
    m = K - ngt
    eq_pre = _prefix_incl(meq.astype(jnp.float32))
    sel = mgt | (meq & (eq_pre.astype(jnp.int32) <= m))
    pos = _prefix_incl(sel.astype(jnp.float32)).astype(jnp.int32) - 1
    pos = jnp.where(sel, pos, -1)

    iota_s = lax.broadcasted_iota(jnp.int32, (1, S), 1)
    hi = ((keys >> 16) + 32768).astype(jnp.float32)
    lo = (keys & 0xFFFF).astype(jnp.float32)
    iota_f = jnp.broadcast_to(iota_s, (B, S)).astype(jnp.float32)

    slot = lax.broadcasted_iota(jnp.int32, (SEL, S), 0)
    for b in range(B):
        a = (pos[b][None, :] == slot).astype(jnp.float32)
        v = jnp.concatenate(
            [iota_f[b][:, None], hi[b][:, None], lo[b][:, None]],
            axis=1)
        cmp = jnp.dot(a, v, precision=lax.Precision.HIGHEST,
                      preferred_element_type=jnp.float32)
        idx_c = cmp[:, 0].astype(jnp.int32)
        hi_c = cmp[:, 1].astype(jnp.int32)
        lo_c = cmp[:, 2].astype(jnp.int32)
        gidx_ref[b, :] = idx_c
        selk_ref[b, :] = ((hi_c - 32768) << 16) | lo_c


def _select(keys):
    return pl.pallas_call(
        _select_body,
        out_shape=[
            jax.ShapeDtypeStruct((B, SEL), jnp.int32),
            jax.ShapeDtypeStruct((B, SEL), jnp.int32),
        ],
    )(keys)


_GC = 32


def _gather_body(gidx_hbm, x_hbm, rows_hbm, idx_v, rows_v, sem):
    c = lax.axis_index("c")
    s = lax.axis_index("s")
    wid = s * 2 + c
    b = wid // 8
    j = wid - b * 8
    st = j * 24
    pltpu.sync_copy(gidx_hbm.at[b], idx_v)
    pltpu.async_copy(x_hbm.at[b].at[idx_v.at[pl.ds(st, _GC)]], rows_v,
                     sem).wait()
    pltpu.sync_copy(rows_v, rows_hbm.at[pl.ds(b * K + st, _GC)])


@functools.partial(
    pl.kernel,
    out_type=jax.ShapeDtypeStruct((B * K, D), jnp.float32),
    mesh=plsc.VectorSubcoreMesh(core_axis_name="c", subcore_axis_name="s",
                                num_cores=2, num_subcores=16),
    scratch_types=[
        pltpu.VMEM((SEL,), jnp.int32),
        pltpu.VMEM((_GC, D), jnp.float32),
        pltpu.SemaphoreType.DMA,
    ],
)
def _sc_gather(gidx_hbm, x_hbm, rows_hbm, idx_v, rows_v, sem):
    _gather_body(gidx_hbm, x_hbm, rows_hbm, idx_v, rows_v, sem)


_BO = 1024


def _proj_body(sel_ref, selk_ref, wp_ref, bp_ref, o_ref):
    bi = pl.program_id(1)
    keys = selk_ref[pl.ds(bi, 1), :]
    kcol = keys.reshape(SEL, 1)
    jpos = lax.broadcasted_iota(jnp.int32, (SEL, SEL), 0)
    ipos = lax.broadcasted_iota(jnp.int32, (SEL, SEL), 1)
    beats = (kcol > keys) | ((kcol == keys) & (jpos < ipos))
    ranks = jnp.sum(beats.astype(jnp.int32), axis=0)[:K]
    rrow = lax.broadcasted_iota(jnp.int32, (K, K), 0)
    perm = (rrow == ranks[None, :]).astype(jnp.float32)
    ordered = jnp.dot(perm, sel_ref[0],
                      precision=lax.Precision.HIGHEST,
                      preferred_element_type=jnp.float32)
    o_ref[0] = (jnp.dot(ordered, wp_ref[...],
                        preferred_element_type=jnp.float32) + bp_ref[...])


def _proj(sel, selk, wpt, bpr):
    return pl.pallas_call(
        _proj_body,
        grid=(OUT // _BO, B),
        in_specs=[
            pl.BlockSpec((1, K, D), lambda o, b: (b, 0, 0)),
            pl.BlockSpec((B, SEL), lambda o, b: (0, 0)),
            pl.BlockSpec((D, _BO), lambda o, b: (0, o)),
            pl.BlockSpec((1, _BO), lambda o, b: (0, o)),
        ],
        out_specs=pl.BlockSpec((1, K, _BO), lambda o, b: (b, 0, o)),
        out_shape=jax.ShapeDtypeStruct((B, K, OUT), jnp.float32),
    )(sel, selk, wpt, bpr)


def kernel(x, W1, b1, W2, b2, Wp, bp):
    if x.ndim == 2:
        x = x[None, :, :]
    keys = _scores(x, W1.T, b1.reshape(1, H), W2.T)
    gidx, selk = _select(keys)
    rows = _sc_gather(gidx, x)
    return _proj(rows.reshape(B, K, D), selk, Wp.T, bp.reshape(1, OUT))


# speedup vs baseline: 1.0064x; 1.0064x over previous
import functools

import jax
import jax.numpy as jnp
import numpy as np
from jax import lax
from jax.experimental import pallas as pl
from jax.experimental.pallas import tpu as pltpu
from jax.experimental.pallas import tpu_sc as plsc

B, S, D = 4, 8192, 768
H = D // 2
OUT = 4096
K = 200
SEL = 208
SR, SC_ = 64, 128
INT_MIN = np.int32(-2**31)


_BS = 512


def _score_body(x_ref, w1_ref, b1_ref, w2_ref, k_ref):
    xb = x_ref[...].reshape(B * _BS, D)
    h = jnp.dot(xb, w1_ref[...], preferred_element_type=jnp.float32)
    h = h + b1_ref[...]
    h = 0.5 * h * (1.0 + lax.erf(h * np.float32(np.sqrt(0.5))))
    s = jnp.dot(h, w2_ref[...], preferred_element_type=jnp.float32)
    bits = lax.bitcast_convert_type(s[:, 0], jnp.int32)
    keys = jnp.where(bits < 0, bits ^ jnp.int32(0x7FFFFFFF), bits)
    k_ref[...] = keys.reshape(B, _BS)


def _scores(x, w1t, b1r, w2t):
    return pl.pallas_call(
        _score_body,
        grid=(S // _BS,),
        in_specs=[
            pl.BlockSpec((B, _BS, D), lambda s: (0, s, 0)),
            pl.BlockSpec((D, H), lambda s: (0, 0)),
            pl.BlockSpec((1, H), lambda s: (0, 0)),
            pl.BlockSpec((H, 1), lambda s: (0, 0)),
        ],
        out_specs=pl.BlockSpec((B, _BS), lambda s: (0, s)),
        out_shape=jax.ShapeDtypeStruct((B, S), jnp.int32),
    )(x, w1t, b1r, w2t)


def _prefix_incl(m):
    m2 = m.reshape(B * SR, SC_)
    tri_c = (lax.broadcasted_iota(jnp.int32, (SC_, SC_), 0)
             <= lax.broadcasted_iota(jnp.int32, (SC_, SC_), 1))
    intra = jnp.dot(m2, tri_c.astype(jnp.float32),
                    precision=lax.Precision.HIGHEST,
                    preferred_element_type=jnp.float32)
    totals = intra[:, SC_ - 1].reshape(B, SR)
    tri_r = (lax.broadcasted_iota(jnp.int32, (SR, SR), 0)
             < lax.broadcasted_iota(jnp.int32, (SR, SR), 1))
    rowpre = jnp.dot(totals, tri_r.astype(jnp.float32),
                     precision=lax.Precision.HIGHEST,
                     preferred_element_type=jnp.float32)
    out = intra.reshape(B, SR, SC_) + rowpre[:, :, None]
    return out.reshape(B, S)


def _select_body(k_ref, gidx_ref):
    keys = k_ref[...]

    def bs_body(i, p):
        cand_u = p | (jnp.int32(1) << (jnp.int32(31) - i))
        cand_s = cand_u ^ INT_MIN
        cnt = jnp.sum((keys >= cand_s).astype(jnp.int32),
                      axis=1, keepdims=True)
        return jnp.where(cnt >= K, cand_u, p)
    t_u = lax.fori_loop(0, 32, bs_body, jnp.zeros((B, 1), jnp.int32))
    t_s = t_u ^ INT_MIN

    mgt = keys > t_s
    meq = keys == t_s
    ngt = jnp.sum(mgt.astype(jnp.int32), axis=1, keepdims=True)
    m = K - ngt
    eq_pre = _prefix_incl(meq.astype(jnp.float32))
    sel = mgt | (meq & (eq_pre.astype(jnp.int32) <= m))
    pos = _prefix_incl(sel.astype(jnp.float32)).astype(jnp.int32) - 1
    pos = jnp.where(sel, pos, -1)

    iota_s = lax.broadcasted_iota(jnp.int32, (1, S), 1)
    hi = ((keys >> 16) + 32768).astype(jnp.float32)
    lo = (keys & 0xFFFF).astype(jnp.float32)
    iota_f = jnp.broadcast_to(iota_s, (B, S)).astype(jnp.float32)

    iota16 = lax.broadcasted_iota(jnp.int32, (1, 16), 1)
    qslot = lax.broadcasted_iota(jnp.int32, (SEL // 16, 1), 0)
    jpos = lax.broadcasted_iota(jnp.int32, (SEL, SEL), 0)
    ipos = lax.broadcasted_iota(jnp.int32, (SEL, SEL), 1)
    rnk_row = lax.broadcasted_iota(jnp.int32, (SEL, SEL), 0)
    prow16 = lax.broadcasted_iota(jnp.int32, (SEL, 16), 0)
    mr = ((prow16 & 15)
          == lax.broadcasted_iota(jnp.int32, (SEL, 16), 1)
          ).astype(jnp.float32)
    prow13 = lax.broadcasted_iota(jnp.int32, (SEL, SEL // 16), 0)
    rq = ((prow13 >> 4)
          == lax.broadcasted_iota(jnp.int32, (SEL, SEL // 16), 1)
          ).astype(jnp.float32)
    for b in range(B):
        pb = pos[b]
        qm = ((pb >> 4)[None, :] == qslot).astype(jnp.float32)
        rm = ((pb & 15)[:, None] == iota16).astype(jnp.float32)
        v = jnp.concatenate(
            [rm * iota_f[b][:, None], rm * hi[b][:, None],
             rm * lo[b][:, None]], axis=1)
        cmp = jnp.dot(qm, v, precision=lax.Precision.HIGHEST,
                      preferred_element_type=jnp.float32)
        z = jnp.dot(rq, cmp, precision=lax.Precision.HIGHEST,
                    preferred_element_type=jnp.float32)
        idx_c = jnp.sum(z[:, 0:16] * mr, axis=1).astype(jnp.int32)
        hi_c = jnp.sum(z[:, 16:32] * mr, axis=1).astype(jnp.int32)
        lo_c = jnp.sum(z[:, 32:48] * mr, axis=1).astype(jnp.int32)
        keys_c = ((hi_c - 32768) << 16) | lo_c

        kcol = keys_c[:, None]
        beats = (kcol > keys_c[None, :]) | ((kcol == keys_c[None, :])
                                            & (jpos < ipos))
        ranks = jnp.sum(beats.astype(jnp.int32), axis=0)
        pmat = (rnk_row == ranks[None, :]).astype(jnp.float32)
        gidx_ref[b, :] = jnp.dot(
            pmat, idx_c.astype(jnp.float32)[:, None],
            precision=lax.Precision.HIGHEST,
            preferred_element_type=jnp.float32)[:, 0].astype(jnp.int32)


def _select(keys):
    return pl.pallas_call(
        _select_body,
        out_shape=jax.ShapeDtypeStruct((B, SEL), jnp.int32),
    )(keys)


_GC = 32


def _gather_body(gidx_hbm, x_hbm, rows_hbm, idx_v, rows_v, sem):
    c = lax.axis_index("c")
    s = lax.axis_index("s")
    wid = s * 2 + c
    b = wid // 8
    j = wid - b * 8
    st = j * 24
    pltpu.sync_copy(gidx_hbm.at[b], idx_v)
    pltpu.async_copy(x_hbm.at[b].at[idx_v.at[pl.ds(st, _GC)]], rows_v,
                     sem).wait()
    pltpu.sync_copy(rows_v, rows_hbm.at[pl.ds(b * K + st, _GC)])


@functools.partial(
    pl.kernel,
    out_type=jax.ShapeDtypeStruct((B * K, D), jnp.float32),
    mesh=plsc.VectorSubcoreMesh(core_axis_name="c", subcore_axis_name="s",
                                num_cores=2, num_subcores=16),
    scratch_types=[
        pltpu.VMEM((SEL,), jnp.int32),
        pltpu.VMEM((_GC, D), jnp.float32),
        pltpu.SemaphoreType.DMA,
    ],
)
def _sc_gather(gidx_hbm, x_hbm, rows_hbm, idx_v, rows_v, sem):
    _gather_body(gidx_hbm, x_hbm, rows_hbm, idx_v, rows_v, sem)


_BO = 1024


def _proj_body(sel_ref, wp_ref, bp_ref, o_ref):
    o_ref[0] = (jnp.dot(sel_ref[0], wp_ref[...],
                        preferred_element_type=jnp.float32) + bp_ref[...])


def _proj(sel, wpt, bpr):
    return pl.pallas_call(
        _proj_body,
        grid=(OUT // _BO, B),
        in_specs=[
            pl.BlockSpec((1, K, D), lambda o, b: (b, 0, 0)),
            pl.BlockSpec((D, _BO), lambda o, b: (0, o)),
            pl.BlockSpec((1, _BO), lambda o, b: (0, o)),
        ],
        out_specs=pl.BlockSpec((1, K, _BO), lambda o, b: (b, 0, o)),
        out_shape=jax.ShapeDtypeStruct((B, K, OUT), jnp.float32),
    )(sel, wpt, bpr)


def kernel(x, W1, b1, W2, b2, Wp, bp):
    if x.ndim == 2:
        x = x[None, :, :]
    keys = _scores(x, W1.T, b1.reshape(1, H), W2.T)
    gidx = _select(keys)
    rows = _sc_gather(gidx, x)
    return _proj(rows.reshape(B, K, D), Wp.T, bp.reshape(1, OUT))


# speedup vs baseline: 1.0744x; 1.0676x over previous
import functools

import jax
import jax.numpy as jnp
import numpy as np
from jax import lax
from jax.experimental import pallas as pl
from jax.experimental.pallas import tpu as pltpu
from jax.experimental.pallas import tpu_sc as plsc

B, S, D = 4, 8192, 768
H = D // 2
OUT = 4096
K = 200
SEL = 208
SR, SC_ = 64, 128
INT_MIN = np.int32(-2**31)


_BS = 1024


def _score_body(x_ref, w1_ref, b1_ref, w2_ref, k_ref):
    xb = x_ref[...].reshape(B * _BS, D)
    h = jnp.dot(xb, w1_ref[...], preferred_element_type=jnp.float32)
    h = h + b1_ref[...]
    h = 0.5 * h * (1.0 + lax.erf(h * np.float32(np.sqrt(0.5))))
    s = jnp.dot(h, w2_ref[...], preferred_element_type=jnp.float32)
    bits = lax.bitcast_convert_type(s[:, 0], jnp.int32)
    keys = jnp.where(bits < 0, bits ^ jnp.int32(0x7FFFFFFF), bits)
    k_ref[...] = keys.reshape(B, _BS)


def _scores(x, w1t, b1r, w2t):
    return pl.pallas_call(
        _score_body,
        grid=(S // _BS,),
        in_specs=[
            pl.BlockSpec((B, _BS, D), lambda s: (0, s, 0)),
            pl.BlockSpec((D, H), lambda s: (0, 0)),
            pl.BlockSpec((1, H), lambda s: (0, 0)),
            pl.BlockSpec((H, 1), lambda s: (0, 0)),
        ],
        out_specs=pl.BlockSpec((B, _BS), lambda s: (0, s)),
        out_shape=jax.ShapeDtypeStruct((B, S), jnp.int32),
    )(x, w1t, b1r, w2t)


def _prefix_incl(m):
    m2 = m.reshape(B * SR, SC_)
    tri_c = (lax.broadcasted_iota(jnp.int32, (SC_, SC_), 0)
             <= lax.broadcasted_iota(jnp.int32, (SC_, SC_), 1))
    intra = jnp.dot(m2, tri_c.astype(jnp.float32),
                    precision=lax.Precision.HIGHEST,
                    preferred_element_type=jnp.float32)
    totals = intra[:, SC_ - 1].reshape(B, SR)
    tri_r = (lax.broadcasted_iota(jnp.int32, (SR, SR), 0)
             < lax.broadcasted_iota(jnp.int32, (SR, SR), 1))
    rowpre = jnp.dot(totals, tri_r.astype(jnp.float32),
                     precision=lax.Precision.HIGHEST,
                     preferred_element_type=jnp.float32)
    out = intra.reshape(B, SR, SC_) + rowpre[:, :, None]
    return out.reshape(B, S)


def _select_body(k_ref, gidx_ref):
    keys = k_ref[...]

    def bs_body(i, p):
        cand_u = p | (jnp.int32(1) << (jnp.int32(31) - i))
        cand_s = cand_u ^ INT_MIN
        cnt = jnp.sum((keys >= cand_s).astype(jnp.int32),
                      axis=1, keepdims=True)
        return jnp.where(cnt >= K, cand_u, p)
    t_u = lax.fori_loop(0, 32, bs_body, jnp.zeros((B, 1), jnp.int32))
    t_s = t_u ^ INT_MIN

    mgt = keys > t_s
    meq = keys == t_s
    ngt = jnp.sum(mgt.astype(jnp.int32), axis=1, keepdims=True)
    m = K - ngt
    eq_pre = _prefix_incl(meq.astype(jnp.float32))
    sel = mgt | (meq & (eq_pre.astype(jnp.int32) <= m))
    pos = _prefix_incl(sel.astype(jnp.float32)).astype(jnp.int32) - 1
    pos = jnp.where(sel, pos, -1)

    iota_s = lax.broadcasted_iota(jnp.int32, (1, S), 1)
    hi = ((keys >> 16) + 32768).astype(jnp.float32)
    lo = (keys & 0xFFFF).astype(jnp.float32)
    iota_f = jnp.broadcast_to(iota_s, (B, S)).astype(jnp.float32)

    iota16 = lax.broadcasted_iota(jnp.int32, (1, 16), 1)
    qslot = lax.broadcasted_iota(jnp.int32, (SEL // 16, 1), 0)
    jpos = lax.broadcasted_iota(jnp.int32, (SEL, SEL), 0)
    ipos = lax.broadcasted_iota(jnp.int32, (SEL, SEL), 1)
    rnk_row = lax.broadcasted_iota(jnp.int32, (SEL, SEL), 0)
    prow16 = lax.broadcasted_iota(jnp.int32, (SEL, 16), 0)
    mr = ((prow16 & 15)
          == lax.broadcasted_iota(jnp.int32, (SEL, 16), 1)
          ).astype(jnp.float32)
    prow13 = lax.broadcasted_iota(jnp.int32, (SEL, SEL // 16), 0)
    rq = ((prow13 >> 4)
          == lax.broadcasted_iota(jnp.int32, (SEL, SEL // 16), 1)
          ).astype(jnp.float32)
    for b in range(B):
        pb = pos[b]
        qm = ((pb >> 4)[None, :] == qslot).astype(jnp.float32)
        rm = ((pb & 15)[:, None] == iota16).astype(jnp.float32)
        v = jnp.concatenate(
            [rm * iota_f[b][:, None], rm * hi[b][:, None],
             rm * lo[b][:, None]], axis=1)
        cmp = jnp.dot(qm, v, precision=lax.Precision.HIGHEST,
                      preferred_element_type=jnp.float32)
        z = jnp.dot(rq, cmp, precision=lax.Precision.HIGHEST,
                    preferred_element_type=jnp.float32)
        idx_c = jnp.sum(z[:, 0:16] * mr, axis=1).astype(jnp.int32)
        hi_c = jnp.sum(z[:, 16:32] * mr, axis=1).astype(jnp.int32)
        lo_c = jnp.sum(z[:, 32:48] * mr, axis=1).astype(jnp.int32)
        keys_c = ((hi_c - 32768) << 16) | lo_c

        kcol = keys_c[:, None]
        beats = (kcol > keys_c[None, :]) | ((kcol == keys_c[None, :])
                                            & (jpos < ipos))
        ranks = jnp.sum(beats.astype(jnp.int32), axis=0)
        pmat = (rnk_row == ranks[None, :]).astype(jnp.float32)
        gidx_ref[b, :] = jnp.dot(
            pmat, idx_c.astype(jnp.float32)[:, None],
            precision=lax.Precision.HIGHEST,
            preferred_element_type=jnp.float32)[:, 0].astype(jnp.int32)


def _select(keys):
    return pl.pallas_call(
        _select_body,
        out_shape=jax.ShapeDtypeStruct((B, SEL), jnp.int32),
    )(keys)


_GC = 32


def _gather_body(gidx_hbm, x_hbm, rows_hbm, idx_v, rows_v, sem):
    c = lax.axis_index("c")
    s = lax.axis_index("s")
    wid = s * 2 + c
    b = wid // 8
    j = wid - b * 8
    st = j * 24
    pltpu.sync_copy(gidx_hbm.at[b], idx_v)
    pltpu.async_copy(x_hbm.at[b].at[idx_v.at[pl.ds(st, _GC)]], rows_v,
                     sem).wait()
    pltpu.sync_copy(rows_v, rows_hbm.at[pl.ds(b * K + st, _GC)])


@functools.partial(
    pl.kernel,
    out_type=jax.ShapeDtypeStruct((B * K, D), jnp.float32),
    mesh=plsc.VectorSubcoreMesh(core_axis_name="c", subcore_axis_name="s",
                                num_cores=2, num_subcores=16),
    scratch_types=[
        pltpu.VMEM((SEL,), jnp.int32),
        pltpu.VMEM((_GC, D), jnp.float32),
        pltpu.SemaphoreType.DMA,
    ],
)
def _sc_gather(gidx_hbm, x_hbm, rows_hbm, idx_v, rows_v, sem):
    _gather_body(gidx_hbm, x_hbm, rows_hbm, idx_v, rows_v, sem)


_BO = 1024


def _proj_body(sel_ref, wp_ref, bp_ref, o_ref):
    o_ref[...] = (jnp.dot(sel_ref[...], wp_ref[...],
                          preferred_element_type=jnp.float32) + bp_ref[...])


def _proj(sel, wpt, bpr):
    return pl.pallas_call(
        _proj_body,
        grid=(OUT // _BO,),
        in_specs=[
            pl.BlockSpec((B * K, D), lambda o: (0, 0)),
            pl.BlockSpec((D, _BO), lambda o: (0, o)),
            pl.BlockSpec((1, _BO), lambda o: (0, o)),
        ],
        out_specs=pl.BlockSpec((B * K, _BO), lambda o: (0, o)),
        out_shape=jax.ShapeDtypeStruct((B * K, OUT), jnp.float32),
    )(sel, wpt, bpr)


def kernel(x, W1, b1, W2, b2, Wp, bp):
    if x.ndim == 2:
        x = x[None, :, :]
    keys = _scores(x, W1.T, b1.reshape(1, H), W2.T)
    gidx = _select(keys)
    rows = _sc_gather(gidx, x)
    return _proj(rows, Wp.T, bp.reshape(1, OUT)).reshape(B, K, OUT)
